# baseline (speedup 1.0000x reference)
import functools

import jax
import jax.numpy as jnp
from jax import lax
from jax.experimental import pallas as pl
from jax.experimental.pallas import tpu as pltpu
from jax.experimental.pallas import tpu_sc as plsc

N, D, H, C = 10000, 128, 128, 40
E = 160000

BLK = 1000
GRID = N // BLK

NTILES = 32
CHUNK = 128
EP = 163840
SROWS = 16
R0 = 80
NP = 10112
OPT = NP // 16


def _mm(a, b):
    return jnp.dot(a, b, preferred_element_type=jnp.float32)


_row_spec = pl.BlockSpec((BLK, H), lambda i: (i, 0))
_p_spec = pl.BlockSpec((2, BLK, H), lambda i: (0, i, 0))
_w_spec = pl.BlockSpec((H, H), lambda i: (0, 0))
_b_spec = pl.BlockSpec((1, H), lambda i: (0, 0))
_wc_spec = pl.BlockSpec((H, C), lambda i: (0, 0))
_bc_spec = pl.BlockSpec((1, C), lambda i: (0, 0))
_out_c_spec = pl.BlockSpec((BLK, C), lambda i: (i, 0))

_f32 = jnp.float32


def _tc1_body(x, win, bin_, wa0, ba0, wb0, bb0, wa1, ba1, wb1, bb1,
              h_o, a0_o, a1_o):
    h = jnp.maximum(_mm(x[...], win[...]) + bin_[...], 0.0)
    h_o[...] = h
    a0_o[...] = _mm(jnp.maximum(_mm(h, wa0[...]) + ba0[...], 0.0), wb0[...]) + bb0[...]
    a1_o[...] = _mm(jnp.maximum(_mm(h, wa1[...]) + ba1[...], 0.0), wb1[...]) + bb1[...]


_tc1 = pl.pallas_call(
    _tc1_body,
    grid=(GRID,),
    in_specs=[_row_spec] + [_w_spec, _b_spec] * 5,
    out_specs=[_row_spec] * 3,
    out_shape=[jax.ShapeDtypeStruct((N, H), _f32)] * 3,
)


def _tc2_body(h, p, uwah, uwag, uba, uwb, ubb,
              wa0, ba0, wb0, bb0, wa1, ba1, wb1, bb1,
              h1_o, a0_o, a1_o):
    agg = p[...]
    t = jnp.maximum(_mm(h[...], uwah[...]) + _mm(agg, uwag[...]) + uba[...], 0.0)
    h1 = _mm(t, uwb[...]) + ubb[...]
    h1_o[...] = h1
    a0_o[...] = _mm(jnp.maximum(_mm(h1, wa0[...]) + ba0[...], 0.0), wb0[...]) + bb0[...]
    a1_o[...] = _mm(jnp.maximum(_mm(h1, wa1[...]) + ba1[...], 0.0), wb1[...]) + bb1[...]


_tc2 = pl.pallas_call(
    _tc2_body,
    grid=(GRID,),
    in_specs=[_row_spec, _row_spec, _w_spec, _w_spec, _b_spec, _w_spec, _b_spec]
             + [_w_spec, _b_spec] * 4,
    out_specs=[_row_spec] * 3,
    out_shape=[jax.ShapeDtypeStruct((N, H), _f32)] * 3,
)


def _tc3_body(h, p, uwah, uwag, uba, uwb, ubb, cwa, cba, cwb, cbb, out_o):
    agg = p[...]
    t = jnp.maximum(_mm(h[...], uwah[...]) + _mm(agg, uwag[...]) + uba[...], 0.0)
    h2 = _mm(t, uwb[...]) + ubb[...]
    out_o[...] = _mm(jnp.maximum(_mm(h2, cwa[...]) + cba[...], 0.0), cwb[...]) + cbb[...]


_tc3 = pl.pallas_call(
    _tc3_body,
    grid=(GRID,),
    in_specs=[_row_spec, _row_spec, _w_spec, _w_spec, _b_spec, _w_spec, _b_spec,
              _w_spec, _b_spec, _wc_spec, _bc_spec],
    out_specs=_out_c_spec,
    out_shape=jax.ShapeDtypeStruct((N, C), _f32),
)


_sc_mesh = plsc.VectorSubcoreMesh(core_axis_name="c", subcore_axis_name="s",
                                  num_cores=1)


@functools.partial(
    pl.kernel,
    out_type=jax.ShapeDtypeStruct((NP, H), _f32),
    mesh=_sc_mesh,
    scratch_types=[
        pltpu.VMEM((SROWS, CHUNK), jnp.int32),
        pltpu.VMEM((SROWS, CHUNK), jnp.int32),
        pltpu.VMEM((CHUNK, H), _f32),
        pltpu.VMEM((CHUNK, H), _f32),
        pltpu.VMEM_SHARED((NP, H), _f32),
        pltpu.SemaphoreType.DMA,
        pltpu.SemaphoreType.DMA,
        pltpu.SemaphoreType.DMA,
        pltpu.SemaphoreType.DMA,
    ],
)
def _sc_agg(a0_hbm, a1_hbm, src0_hbm, dst0_hbm, src1_hbm, dst1_hbm,
            out_hbm, sidx_v, didx_v,
            rows0, rows1, acc, gsem0, gsem1, ssem0, ssem1):
    rows = (rows0, rows1)
    gsem = (gsem0, gsem1)
    ssem = (ssem0, ssem1)
    c = lax.axis_index("c")
    s = lax.axis_index("s")

    def _zero_row(i, carry):
        for j in range(H // 16):
            rows[0][i, pl.ds(j * 16, 16)] = jnp.zeros((16,), _f32)
        return carry

    lax.fori_loop(0, CHUNK, _zero_row, 0)
    for k in range(OPT // CHUNK):
        pltpu.sync_copy(rows[0], acc.at[pl.ds(s * OPT + k * CHUNK, CHUNK)])
    rem = OPT % CHUNK
    if rem:
        pltpu.sync_copy(rows[0].at[pl.ds(0, rem)],
                        acc.at[pl.ds(s * OPT + OPT - rem, rem)])
    plsc.subcore_barrier()

    def _stage(a_hbm, src_hbm, dst_hbm, stage_base, srows):
        pltpu.sync_copy(src_hbm.at[pl.ds(stage_base, srows)],
                        sidx_v.at[pl.ds(0, srows)])
        pltpu.sync_copy(dst_hbm.at[pl.ds(stage_base, srows)],
                        didx_v.at[pl.ds(0, srows)])
        pltpu.async_copy(a_hbm.at[sidx_v.at[0]], rows[0], gsem[0])

        def pair(ci, carry):
            for b in range(2):
                chunk = ci * 2 + b
                pltpu.make_async_copy(a_hbm.at[sidx_v.at[chunk]],
                                      rows[b], gsem[b]).wait()
                pltpu.async_copy(rows[b], acc.at[didx_v.at[chunk]], ssem[b],
                                 add=True)
                nxt = chunk + 1

                @pl.when(nxt < srows)
                def _():
                    @pl.when(chunk >= 1)
                    def _():
                        pltpu.make_async_copy(
                            rows[1 - b], acc.at[didx_v.at[chunk - 1]],
                            ssem[1 - b]).wait()

                    pltpu.async_copy(a_hbm.at[sidx_v.at[nxt]],
                                     rows[1 - b], gsem[1 - b])
            return carry

        lax.fori_loop(0, srows // 2, pair, 0)
        pltpu.make_async_copy(rows[0], acc.at[didx_v.at[srows - 2]],
                              ssem[0]).wait()
        pltpu.make_async_copy(rows[1], acc.at[didx_v.at[srows - 1]],
                              ssem[1]).wait()

    def _edge_pass(a_hbm, src_hbm, dst_hbm):
        for k in range(R0 // SROWS):
            _stage(a_hbm, src_hbm, dst_hbm, s * R0 + k * SROWS, SROWS)

    _edge_pass(a0_hbm, src0_hbm, dst0_hbm)
    _edge_pass(a1_hbm, src1_hbm, dst1_hbm)
    plsc.subcore_barrier()

    pltpu.sync_copy(acc.at[pl.ds(s * OPT, OPT)],
                    out_hbm.at[pl.ds(s * OPT, OPT)])


def _pad_edges(ei):
    pad = EP - E
    src = jnp.concatenate([ei[0], jnp.zeros((pad,), jnp.int32)])
    dst = jnp.concatenate([ei[1], jnp.full((pad,), NP - 1, jnp.int32)])
    return src.reshape(EP // CHUNK, CHUNK), dst.reshape(EP // CHUNK, CHUNK)


def kernel(x, ei_r0, ei_r1, params):
    p = params
    zf = (p["la_feat"] >= 0.0).astype(_f32)
    win = p["Win"] * zf[:, None]
    bin_ = p["bin"].reshape(1, H)
    g0 = (p["la_r0"] >= 0.0).astype(_f32)[0]
    g1 = (p["la_r1"] >= 0.0).astype(_f32)[0]
    wa0, ba0 = p["r0_Wa"], p["r0_ba"].reshape(1, H)
    wb0, bb0 = p["r0_Wb"] * g0, (p["r0_bb"] * g0).reshape(1, H)
    wa1, ba1 = p["r1_Wa"], p["r1_ba"].reshape(1, H)
    wb1, bb1 = p["r1_Wb"] * g1, (p["r1_bb"] * g1).reshape(1, H)

    u = []
    for l in range(2):
        uw = p["U%d_Wa" % l]
        u.append((uw[:H], uw[H:], p["U%d_ba" % l].reshape(1, H),
                  p["U%d_Wb" % l], p["U%d_bb" % l].reshape(1, H)))
    cwa, cba = p["C_Wa"], p["C_ba"].reshape(1, H)
    cwb, cbb = p["C_Wb"], p["C_bb"].reshape(1, C)

    src0, dst0 = _pad_edges(ei_r0)
    src1, dst1 = _pad_edges(ei_r1)

    et_w = (wa0, ba0, wb0, bb0, wa1, ba1, wb1, bb1)
    h, a0, a1 = _tc1(x, win, bin_, *et_w)
    p1 = _sc_agg(a0, a1, src0, dst0, src1, dst1)
    h1, b0, b1 = _tc2(h, p1, *u[0], *et_w)
    p2 = _sc_agg(b0, b1, src0, dst0, src1, dst1)
    logits = _tc3(h1, p2, *u[1], cwa, cba, cwb, cbb)
    return logits


# speedup vs baseline: 1.5388x; 1.5388x over previous
import functools

import jax
import jax.numpy as jnp
from jax import lax
from jax.experimental import pallas as pl
from jax.experimental.pallas import tpu as pltpu
from jax.experimental.pallas import tpu_sc as plsc

N, D, H, C = 10000, 128, 128, 40
E = 160000

BLK = 1000
GRID = N // BLK

NTILES = 32
CHUNK = 128
EP = 163840
SROWS = 24
R0 = 72
R1 = 8
NP = 10112
OPT = NP // 16


def _mm(a, b):
    return jnp.dot(a, b, preferred_element_type=jnp.float32)


_row_spec = pl.BlockSpec((BLK, H), lambda i: (i, 0))
_p_spec = pl.BlockSpec((2, BLK, H), lambda i: (0, i, 0))
_w_spec = pl.BlockSpec((H, H), lambda i: (0, 0))
_b_spec = pl.BlockSpec((1, H), lambda i: (0, 0))
_wc_spec = pl.BlockSpec((H, C), lambda i: (0, 0))
_bc_spec = pl.BlockSpec((1, C), lambda i: (0, 0))
_out_c_spec = pl.BlockSpec((BLK, C), lambda i: (i, 0))

_f32 = jnp.float32


def _tc1_body(x, win, bin_, wa0, ba0, wb0, bb0, wa1, ba1, wb1, bb1,
              h_o, a0_o, a1_o):
    h = jnp.maximum(_mm(x[...], win[...]) + bin_[...], 0.0)
    h_o[...] = h
    a0_o[...] = _mm(jnp.maximum(_mm(h, wa0[...]) + ba0[...], 0.0), wb0[...]) + bb0[...]
    a1_o[...] = _mm(jnp.maximum(_mm(h, wa1[...]) + ba1[...], 0.0), wb1[...]) + bb1[...]


_tc1 = pl.pallas_call(
    _tc1_body,
    grid=(GRID,),
    in_specs=[_row_spec] + [_w_spec, _b_spec] * 5,
    out_specs=[_row_spec] * 3,
    out_shape=[jax.ShapeDtypeStruct((N, H), _f32)] * 3,
)


def _tc2_body(h, p, uwah, uwag, uba, uwb, ubb,
              wa0, ba0, wb0, bb0, wa1, ba1, wb1, bb1,
              h1_o, a0_o, a1_o):
    agg = p[0] + p[1]
    t = jnp.maximum(_mm(h[...], uwah[...]) + _mm(agg, uwag[...]) + uba[...], 0.0)
    h1 = _mm(t, uwb[...]) + ubb[...]
    h1_o[...] = h1
    a0_o[...] = _mm(jnp.maximum(_mm(h1, wa0[...]) + ba0[...], 0.0), wb0[...]) + bb0[...]
    a1_o[...] = _mm(jnp.maximum(_mm(h1, wa1[...]) + ba1[...], 0.0), wb1[...]) + bb1[...]


_tc2 = pl.pallas_call(
    _tc2_body,
    grid=(GRID,),
    in_specs=[_row_spec, _p_spec, _w_spec, _w_spec, _b_spec, _w_spec, _b_spec]
             + [_w_spec, _b_spec] * 4,
    out_specs=[_row_spec] * 3,
    out_shape=[jax.ShapeDtypeStruct((N, H), _f32)] * 3,
)


def _tc3_body(h, p, uwah, uwag, uba, uwb, ubb, cwa, cba, cwb, cbb, out_o):
    agg = p[0] + p[1]
    t = jnp.maximum(_mm(h[...], uwah[...]) + _mm(agg, uwag[...]) + uba[...], 0.0)
    h2 = _mm(t, uwb[...]) + ubb[...]
    out_o[...] = _mm(jnp.maximum(_mm(h2, cwa[...]) + cba[...], 0.0), cwb[...]) + cbb[...]


_tc3 = pl.pallas_call(
    _tc3_body,
    grid=(GRID,),
    in_specs=[_row_spec, _p_spec, _w_spec, _w_spec, _b_spec, _w_spec, _b_spec,
              _w_spec, _b_spec, _wc_spec, _bc_spec],
    out_specs=_out_c_spec,
    out_shape=jax.ShapeDtypeStruct((N, C), _f32),
)


_sc_mesh = plsc.VectorSubcoreMesh(core_axis_name="c", subcore_axis_name="s")


@functools.partial(
    pl.kernel,
    out_type=jax.ShapeDtypeStruct((2, NP, H), _f32),
    mesh=_sc_mesh,
    scratch_types=[
        pltpu.VMEM((SROWS, CHUNK), jnp.int32),
        pltpu.VMEM((SROWS, CHUNK), jnp.int32),
        pltpu.VMEM((CHUNK, H), _f32),
        pltpu.VMEM((CHUNK, H), _f32),
        pltpu.VMEM_SHARED((NP, H), _f32),
        pltpu.SemaphoreType.DMA,
        pltpu.SemaphoreType.DMA,
        pltpu.SemaphoreType.DMA,
        pltpu.SemaphoreType.DMA,
    ],
)
def _sc_agg(a0_hbm, a1_hbm, src0_hbm, dst0_hbm, src1_hbm, dst1_hbm,
            out_hbm, sidx_v, didx_v,
            rows0, rows1, acc, gsem0, gsem1, ssem0, ssem1):
    rows = (rows0, rows1)
    gsem = (gsem0, gsem1)
    ssem = (ssem0, ssem1)
    c = lax.axis_index("c")
    s = lax.axis_index("s")

    def _zero_row(i, carry):
        for j in range(H // 16):
            rows[0][i, pl.ds(j * 16, 16)] = jnp.zeros((16,), _f32)
        return carry

    lax.fori_loop(0, CHUNK, _zero_row, 0)
    for k in range(OPT // CHUNK):
        pltpu.sync_copy(rows[0], acc.at[pl.ds(s * OPT + k * CHUNK, CHUNK)])
    rem = OPT % CHUNK
    if rem:
        pltpu.sync_copy(rows[0].at[pl.ds(0, rem)],
                        acc.at[pl.ds(s * OPT + OPT - rem, rem)])
    plsc.subcore_barrier()

    def _stage(a_hbm, src_hbm, dst_hbm, stage_base, srows):
        pltpu.sync_copy(src_hbm.at[pl.ds(stage_base, srows)],
                        sidx_v.at[pl.ds(0, srows)])
        pltpu.sync_copy(dst_hbm.at[pl.ds(stage_base, srows)],
                        didx_v.at[pl.ds(0, srows)])
        pltpu.async_copy(a_hbm.at[sidx_v.at[0]], rows[0], gsem[0])

        def pair(ci, carry):
            for b in range(2):
                chunk = ci * 2 + b
                pltpu.make_async_copy(a_hbm.at[sidx_v.at[chunk]],
                                      rows[b], gsem[b]).wait()
                pltpu.async_copy(rows[b], acc.at[didx_v.at[chunk]], ssem[b],
                                 add=True)
                nxt = chunk + 1

                @pl.when(nxt < srows)
                def _():
                    @pl.when(chunk >= 1)
                    def _():
                        pltpu.make_async_copy(
                            rows[1 - b], acc.at[didx_v.at[chunk - 1]],
                            ssem[1 - b]).wait()

                    pltpu.async_copy(a_hbm.at[sidx_v.at[nxt]],
                                     rows[1 - b], gsem[1 - b])
            return carry

        lax.fori_loop(0, srows // 2, pair, 0)
        pltpu.make_async_copy(rows[0], acc.at[didx_v.at[srows - 2]],
                              ssem[0]).wait()
        pltpu.make_async_copy(rows[1], acc.at[didx_v.at[srows - 1]],
                              ssem[1]).wait()

    def _edge_pass(a_hbm, src_hbm, dst_hbm):
        @pl.when(c == 0)
        def _():
            for k in range(R0 // SROWS):
                _stage(a_hbm, src_hbm, dst_hbm, s * R0 + k * SROWS, SROWS)

        @pl.when(c == 1)
        def _():
            _stage(a_hbm, src_hbm, dst_hbm, 16 * R0 + s * R1, R1)

    _edge_pass(a0_hbm, src0_hbm, dst0_hbm)
    _edge_pass(a1_hbm, src1_hbm, dst1_hbm)
    plsc.subcore_barrier()

    pltpu.sync_copy(acc.at[pl.ds(s * OPT, OPT)],
                    out_hbm.at[c, pl.ds(s * OPT, OPT)])


def _pad_edges(ei):
    pad = EP - E
    src = jnp.concatenate([ei[0], jnp.zeros((pad,), jnp.int32)])
    dst = jnp.concatenate([ei[1], jnp.full((pad,), NP - 1, jnp.int32)])
    return src.reshape(EP // CHUNK, CHUNK), dst.reshape(EP // CHUNK, CHUNK)


def kernel(x, ei_r0, ei_r1, params):
    p = params
    zf = (p["la_feat"] >= 0.0).astype(_f32)
    win = p["Win"] * zf[:, None]
    bin_ = p["bin"].reshape(1, H)
    g0 = (p["la_r0"] >= 0.0).astype(_f32)[0]
    g1 = (p["la_r1"] >= 0.0).astype(_f32)[0]
    wa0, ba0 = p["r0_Wa"], p["r0_ba"].reshape(1, H)
    wb0, bb0 = p["r0_Wb"] * g0, (p["r0_bb"] * g0).reshape(1, H)
    wa1, ba1 = p["r1_Wa"], p["r1_ba"].reshape(1, H)
    wb1, bb1 = p["r1_Wb"] * g1, (p["r1_bb"] * g1).reshape(1, H)

    u = []
    for l in range(2):
        uw = p["U%d_Wa" % l]
        u.append((uw[:H], uw[H:], p["U%d_ba" % l].reshape(1, H),
                  p["U%d_Wb" % l], p["U%d_bb" % l].reshape(1, H)))
    cwa, cba = p["C_Wa"], p["C_ba"].reshape(1, H)
    cwb, cbb = p["C_Wb"], p["C_bb"].reshape(1, C)

    src0, dst0 = _pad_edges(ei_r0)
    src1, dst1 = _pad_edges(ei_r1)

    et_w = (wa0, ba0, wb0, bb0, wa1, ba1, wb1, bb1)
    h, a0, a1 = _tc1(x, win, bin_, *et_w)
    p1 = _sc_agg(a0, a1, src0, dst0, src1, dst1)
    h1, b0, b1 = _tc2(h, p1, *u[0], *et_w)
    p2 = _sc_agg(b0, b1, src0, dst0, src1, dst1)
    logits = _tc3(h1, p2, *u[1], cwa, cba, cwb, cbb)
    return logits
